# baseline (speedup 1.0000x reference)
import functools

import jax
import jax.numpy as jnp
from jax import lax
from jax.experimental import pallas as pl
from jax.experimental.pallas import tpu as pltpu
from jax.experimental.pallas import tpu_sc as plsc

NC = 2
NS = 16
NW = NC * NS

BATCH = 16384
NUM = 1000000
D = 64
DP = 128
BPW = BATCH // NW
GROUP = 64
NGROUP = BPW // GROUP
L = 16
NSLOT = 12
TAIL = (NUM // DP) * DP
LASTA = TAIL - DP


def _extract_col(src_ref, col, rowbuf, row):
  lanes = lax.iota(jnp.int32, L)
  colv = jnp.full((L,), col, jnp.int32)
  rowv = jnp.full((L,), row, jnp.int32)
  for q in range(D // L):
    jvec = lanes + (q * L)
    vals = plsc.load_gather(src_ref, [jvec, colv])
    plsc.store_scatter(rowbuf, [rowv, lanes + (q * L)], vals)


def _sc_gather_body(uidx_hbm, iidx_hbm, ug_hbm, ig_hbm, um_hbm, im_hbm,
                    tails_hbm,
                    ug_out, ig_out, um_out, im_out,
                    uidx_s, iidx_s, slots, rowbuf, tails, sems):
  c = lax.axis_index("c")
  s = lax.axis_index("s")
  wid = s * NC + c
  base = wid * BPW

  pltpu.sync_copy(uidx_hbm.at[wid], uidx_s.at[pl.ds(0, BPW)])
  pltpu.sync_copy(iidx_hbm.at[wid], iidx_s.at[pl.ds(0, BPW)])
  pltpu.sync_copy(tails_hbm, tails)

  for tnum, (table, is_user, out) in enumerate((
      (ug_hbm, True, ug_out), (ig_hbm, False, ig_out),
      (um_hbm, True, um_out), (im_hbm, False, im_out))):
    idx_s = uidx_s if is_user else iidx_s
    tail_col0 = tnum * D

    def _blk(idx):
      return pl.multiple_of(
          jnp.minimum(idx & jnp.int32(~(DP - 1)), jnp.int32(LASTA)), DP)

    def _idx_at(p):
      return idx_s[pl.ds(p, L)][0]

    def group_body(h, _):
      off = h * GROUP

      for r in range(NSLOT):
        pltpu.async_copy(
            table.at[:, pl.ds(_blk(_idx_at(off + r)), DP)],
            slots.at[r], sems.at[r])

      def row_body(r, _):
        rm = lax.rem(r, NSLOT)
        u = _idx_at(off + r)
        st = _blk(u)
        pltpu.make_async_copy(
            table.at[:, pl.ds(st, DP)], slots.at[rm], sems.at[rm]).wait()

        @pl.when(u < TAIL)
        def _():
          _extract_col(slots.at[rm], u - st, rowbuf, r)

        @pl.when(u >= TAIL)
        def _():
          _extract_col(tails, tail_col0 + (u - TAIL), rowbuf, r)

        @pl.when(r + NSLOT < GROUP)
        def _():
          nu = _idx_at(off + r + NSLOT)
          pltpu.async_copy(
              table.at[:, pl.ds(_blk(nu), DP)], slots.at[rm], sems.at[rm])
        return 0

      lax.fori_loop(0, GROUP, row_body, 0)
      pltpu.sync_copy(
          rowbuf, out.at[pl.ds(pl.multiple_of(base + off, 8), GROUP)])
      return 0

    lax.fori_loop(0, NGROUP, group_body, 0)


@jax.jit
def _sc_gather(uidx, iidx, ue_gmf, ie_gmf, ue_mlp, ie_mlp):
  mesh = plsc.VectorSubcoreMesh(core_axis_name="c", subcore_axis_name="s")
  f32 = jnp.float32
  out_type = [jax.ShapeDtypeStruct((BATCH, DP), f32) for _ in range(4)]
  scratch = [
      pltpu.VMEM((BPW + L,), jnp.int32),
      pltpu.VMEM((BPW + L,), jnp.int32),
      pltpu.VMEM((NSLOT, D, DP), f32),
      pltpu.VMEM((GROUP, DP), f32),
      pltpu.VMEM((D, 4 * D), f32),
      pltpu.SemaphoreType.DMA((NSLOT,)),
  ]
  fn = functools.partial(
      pl.kernel, mesh=mesh, out_type=out_type, scratch_types=scratch,
      compiler_params=pltpu.CompilerParams(use_tc_tiling_on_sc=True,
                                           needs_layout_passes=False),
  )(_sc_gather_body)
  tts = [jnp.swapaxes(t, 0, 1) for t in (ue_gmf, ie_gmf, ue_mlp, ie_mlp)]
  tails = jnp.concatenate(
      [lax.slice(t, (0, TAIL), (D, NUM)) for t in tts], axis=1)
  return fn(uidx.reshape(NW, BPW), iidx.reshape(NW, BPW), *tts, tails)


def _tc_mlp_body(ug_ref, ig_ref, um_ref, im_ref, w1u_ref, w1i_ref, b1_ref,
                 w2_ref, b2_ref, w3_ref, b3_ref, wpg_ref, wph_ref, bp_ref,
                 out_ref):
  hp = jax.lax.Precision.HIGHEST
  um = um_ref[:, :D]
  im = im_ref[:, :D]
  h = jnp.dot(um, w1u_ref[...], precision=hp) + jnp.dot(
      im, w1i_ref[...], precision=hp) + b1_ref[...]
  h = jnp.maximum(h, 0.0)
  h = jnp.maximum(jnp.dot(h, w2_ref[...], precision=hp) + b2_ref[...], 0.0)
  h = jnp.maximum(jnp.dot(h, w3_ref[...], precision=hp) + b3_ref[...], 0.0)
  gmf = ug_ref[:, :D] * ig_ref[:, :D]
  pred = (jnp.dot(gmf, wpg_ref[...], precision=hp)
          + jnp.dot(h, wph_ref[...], precision=hp) + bp_ref[...])
  out_ref[...] = pred


BLK = 2048


@jax.jit
def _tc_mlp(ug, ig, um, im, W1, b1, W2, b2, W3, b3, Wp, bp):
  w1u = W1[:D]
  w1i = W1[D:]
  wpg = Wp[:D]
  wph = Wp[D:]
  b1r = b1.reshape(1, -1)
  b2r = b2.reshape(1, -1)
  b3r = b3.reshape(1, -1)
  bpr = bp.reshape(1, 1)
  grid = (BATCH // BLK,)
  gath_spec = pl.BlockSpec((BLK, DP), lambda i: (i, 0))
  full = lambda a: pl.BlockSpec(a.shape, lambda i: (0,) * a.ndim)
  out = pl.pallas_call(
      _tc_mlp_body,
      grid=grid,
      in_specs=[
          gath_spec, gath_spec, gath_spec, gath_spec,
          full(w1u), full(w1i), full(b1r),
          full(W2), full(b2r), full(W3), full(b3r),
          full(wpg), full(wph), full(bpr),
      ],
      out_specs=pl.BlockSpec((BLK, 1), lambda i: (i, 0)),
      out_shape=jax.ShapeDtypeStruct((BATCH, 1), jnp.float32),
  )(ug, ig, um, im, w1u, w1i, b1r, W2, b2r, W3, b3r, wpg, wph, bpr)
  return out


def kernel(user_indices, item_indices, ue_gmf, ie_gmf, ue_mlp, ie_mlp,
           W1, b1, W2, b2, W3, b3, Wp, bp):
  ug, ig, um, im = _sc_gather(user_indices, item_indices,
                              ue_gmf, ie_gmf, ue_mlp, ie_mlp)
  out = _tc_mlp(ug, ig, um, im, W1, b1, W2, b2, W3, b3, Wp, bp)
  return jnp.squeeze(out, axis=-1)


# speedup vs baseline: 1.0135x; 1.0135x over previous
import functools

import jax
import jax.numpy as jnp
from jax import lax
from jax.experimental import pallas as pl
from jax.experimental.pallas import tpu as pltpu
from jax.experimental.pallas import tpu_sc as plsc

NC = 2
NS = 16
NW = NC * NS

BATCH = 16384
NUM = 1000000
D = 64
DP = 128
BPW = BATCH // NW
GROUP = 64
NGROUP = BPW // GROUP
L = 16
NSLOT = 10
TAIL = (NUM // DP) * DP
LASTA = TAIL - DP


def _extract_col(src_ref, col, rowbuf, row):
  lanes = lax.iota(jnp.int32, L)
  colv = jnp.full((L,), col, jnp.int32)
  rowv = jnp.full((L,), row, jnp.int32)
  for q in range(D // L):
    jvec = lanes + (q * L)
    vals = plsc.load_gather(src_ref, [jvec, colv])
    plsc.store_scatter(rowbuf, [rowv, lanes + (q * L)], vals)


def _sc_gather_body(uidx_hbm, iidx_hbm, ug_hbm, ig_hbm, um_hbm, im_hbm,
                    tails_hbm,
                    ug_out, ig_out, um_out, im_out,
                    uidx_s, iidx_s, slots, rowbuf, tails, sems):
  c = lax.axis_index("c")
  s = lax.axis_index("s")
  wid = s * NC + c
  base = wid * BPW

  pltpu.sync_copy(uidx_hbm.at[wid], uidx_s.at[pl.ds(0, BPW)])
  pltpu.sync_copy(iidx_hbm.at[wid], iidx_s.at[pl.ds(0, BPW)])
  pltpu.sync_copy(tails_hbm, tails)

  for tnum, (table, is_user, out) in enumerate((
      (ug_hbm, True, ug_out), (ig_hbm, False, ig_out),
      (um_hbm, True, um_out), (im_hbm, False, im_out))):
    idx_s = uidx_s if is_user else iidx_s
    tail_col0 = tnum * D

    def _blk(idx):
      return pl.multiple_of(
          jnp.minimum(idx & jnp.int32(~(DP - 1)), jnp.int32(LASTA)), DP)

    def _idx_at(p):
      return idx_s[pl.ds(p, L)][0]

    def group_body(h, _):
      off = h * GROUP

      for r in range(NSLOT):
        pltpu.async_copy(
            table.at[:, pl.ds(_blk(_idx_at(off + r)), DP)],
            slots.at[r], sems.at[r])

      def row_body(r, _):
        rm = lax.rem(r, NSLOT)
        u = _idx_at(off + r)
        st = _blk(u)
        pltpu.make_async_copy(
            table.at[:, pl.ds(st, DP)], slots.at[rm], sems.at[rm]).wait()

        @pl.when(u < TAIL)
        def _():
          _extract_col(slots.at[rm], u - st, rowbuf, r)

        @pl.when(u >= TAIL)
        def _():
          _extract_col(tails, tail_col0 + (u - TAIL), rowbuf, r)

        @pl.when(r + NSLOT < GROUP)
        def _():
          nu = _idx_at(off + r + NSLOT)
          pltpu.async_copy(
              table.at[:, pl.ds(_blk(nu), DP)], slots.at[rm], sems.at[rm])
        return 0

      lax.fori_loop(0, GROUP, row_body, 0)
      pltpu.sync_copy(
          rowbuf, out.at[pl.ds(pl.multiple_of(base + off, 8), GROUP)])
      return 0

    lax.fori_loop(0, NGROUP, group_body, 0)


@jax.jit
def _sc_gather(uidx, iidx, ue_gmf, ie_gmf, ue_mlp, ie_mlp):
  mesh = plsc.VectorSubcoreMesh(core_axis_name="c", subcore_axis_name="s")
  f32 = jnp.float32
  out_type = [jax.ShapeDtypeStruct((BATCH, DP), f32) for _ in range(4)]
  scratch = [
      pltpu.VMEM((BPW + L,), jnp.int32),
      pltpu.VMEM((BPW + L,), jnp.int32),
      pltpu.VMEM((NSLOT, D, DP), f32),
      pltpu.VMEM((GROUP, DP), f32),
      pltpu.VMEM((D, 4 * D), f32),
      pltpu.SemaphoreType.DMA((NSLOT,)),
  ]
  fn = functools.partial(
      pl.kernel, mesh=mesh, out_type=out_type, scratch_types=scratch,
      compiler_params=pltpu.CompilerParams(use_tc_tiling_on_sc=True,
                                           needs_layout_passes=False),
  )(_sc_gather_body)
  tts = [jnp.swapaxes(t, 0, 1) for t in (ue_gmf, ie_gmf, ue_mlp, ie_mlp)]
  tails = jnp.concatenate(
      [lax.slice(t, (0, TAIL), (D, NUM)) for t in tts], axis=1)
  return fn(uidx.reshape(NW, BPW), iidx.reshape(NW, BPW), *tts, tails)


def _tc_mlp_body(ug_ref, ig_ref, um_ref, im_ref, w1u_ref, w1i_ref, b1_ref,
                 w2_ref, b2_ref, w3_ref, b3_ref, wpg_ref, wph_ref, bp_ref,
                 out_ref):
  hp = jax.lax.Precision.HIGHEST
  um = um_ref[:, :D]
  im = im_ref[:, :D]
  h = jnp.dot(um, w1u_ref[...], precision=hp) + jnp.dot(
      im, w1i_ref[...], precision=hp) + b1_ref[...]
  h = jnp.maximum(h, 0.0)
  h = jnp.maximum(jnp.dot(h, w2_ref[...], precision=hp) + b2_ref[...], 0.0)
  h = jnp.maximum(jnp.dot(h, w3_ref[...], precision=hp) + b3_ref[...], 0.0)
  gmf = ug_ref[:, :D] * ig_ref[:, :D]
  pred = (jnp.dot(gmf, wpg_ref[...], precision=hp)
          + jnp.dot(h, wph_ref[...], precision=hp) + bp_ref[...])
  out_ref[...] = pred


BLK = 2048


@jax.jit
def _tc_mlp(ug, ig, um, im, W1, b1, W2, b2, W3, b3, Wp, bp):
  w1u = W1[:D]
  w1i = W1[D:]
  wpg = Wp[:D]
  wph = Wp[D:]
  b1r = b1.reshape(1, -1)
  b2r = b2.reshape(1, -1)
  b3r = b3.reshape(1, -1)
  bpr = bp.reshape(1, 1)
  grid = (BATCH // BLK,)
  gath_spec = pl.BlockSpec((BLK, DP), lambda i: (i, 0))
  full = lambda a: pl.BlockSpec(a.shape, lambda i: (0,) * a.ndim)
  out = pl.pallas_call(
      _tc_mlp_body,
      grid=grid,
      in_specs=[
          gath_spec, gath_spec, gath_spec, gath_spec,
          full(w1u), full(w1i), full(b1r),
          full(W2), full(b2r), full(W3), full(b3r),
          full(wpg), full(wph), full(bpr),
      ],
      out_specs=pl.BlockSpec((BLK, 1), lambda i: (i, 0)),
      out_shape=jax.ShapeDtypeStruct((BATCH, 1), jnp.float32),
  )(ug, ig, um, im, w1u, w1i, b1r, W2, b2r, W3, b3r, wpg, wph, bpr)
  return out


def kernel(user_indices, item_indices, ue_gmf, ie_gmf, ue_mlp, ie_mlp,
           W1, b1, W2, b2, W3, b3, Wp, bp):
  ug, ig, um, im = _sc_gather(user_indices, item_indices,
                              ue_gmf, ie_gmf, ue_mlp, ie_mlp)
  out = _tc_mlp(ug, ig, um, im, W1, b1, W2, b2, W3, b3, Wp, bp)
  return jnp.squeeze(out, axis=-1)
